# Initial kernel scaffold; baseline (speedup 1.0000x reference)
import jax
import jax.numpy as jnp
from jax.experimental import pallas as pl


def kernel(p, x, x_in, weights, kernel_points):
    raise NotImplementedError("write your pallas kernel here")


# speedup vs baseline: 21.4303x; 21.4303x over previous
import functools

import jax
import jax.numpy as jnp
from jax import lax
from jax.experimental import pallas as pl
from jax.experimental.pallas import tpu as pltpu
from jax.experimental.pallas import tpu_sc as plsc

B, N, C_IN, C_OUT = 2, 4096, 64, 64
K = 15
RADIUS = 2.5
NSAMPLE = 16
KP_EXTENT = 1.2

NSUB = 16
PTS_PER_SUB = N // NSUB
GRP = 8
NGRP = PTS_PER_SUB // GRP
NCHUNK = N // 16
PAD_XYZ = 1000000.0
CF = 2 * C_IN


_SC_MESH = plsc.VectorSubcoreMesh(core_axis_name="c", subcore_axis_name="s")


def _sc_body(px, py, pz, xcat,
             nidx_o, relx_o, rely_o, relz_o, fx_o,
             pxv, pyv, pzv, idxbuf,
             st_nidx, st_relx, st_rely, st_relz,
             gidx, rows, sem):
    b = lax.axis_index("c")
    w = lax.axis_index("s")
    bN = b * N
    pltpu.sync_copy(px.at[pl.ds(bN, N)], pxv)
    pltpu.sync_copy(py.at[pl.ds(bN, N)], pyv)
    pltpu.sync_copy(pz.at[pl.ds(bN, N)], pzv)
    base_local = w * PTS_PER_SUB
    r2 = jnp.float32(RADIUS * RADIUS)
    lanes = lax.iota(jnp.int32, 16)

    def point_body(t, carry):
        i_loc = base_local + carry + t
        isplat = jnp.full((16,), i_loc, jnp.int32)
        qx = plsc.load_gather(pxv, [isplat])
        qy = plsc.load_gather(pyv, [isplat])
        qz = plsc.load_gather(pzv, [isplat])

        def scan_cond(st):
            j, cnt = st
            return jnp.logical_and(cnt < NSAMPLE, j < NCHUNK)

        def scan_body(st):
            j, cnt = st
            off = j * 16
            jv = lanes + off
            dx = pxv[pl.ds(off, 16)] - qx
            dy = pyv[pl.ds(off, 16)] - qy
            dz = pzv[pl.ds(off, 16)] - qz
            sq = dx * dx + dy * dy + dz * dz
            m = sq <= r2
            plsc.store_compressed(idxbuf.at[pl.ds(cnt, 16)], jv, mask=m)
            return j + 1, cnt + jnp.sum(m.astype(jnp.int32))

        _, cnt = lax.while_loop(scan_cond, scan_body,
                                (jnp.int32(0), jnp.int32(0)))
        f = jnp.minimum(cnt, NSAMPLE)
        idx16 = idxbuf[pl.ds(0, 16)]
        first = idx16[0]
        valid = lanes < f
        idxv = jnp.where(valid, idx16, first)
        gx = plsc.load_gather(pxv, [idxv])
        gy = plsc.load_gather(pyv, [idxv])
        gz = plsc.load_gather(pzv, [idxv])
        big = jnp.float32(PAD_XYZ)
        st_nidx[t, :] = idxv
        st_relx[t, :] = jnp.where(valid, gx - qx, big)
        st_rely[t, :] = jnp.where(valid, gy - qy, big)
        st_relz[t, :] = jnp.where(valid, gz - qz, big)
        gidx[pl.ds(t * 16, 16)] = idxv + bN
        return carry

    def group_body(g, _):
        lax.fori_loop(0, GRP, point_body, g * GRP)
        cp = pltpu.async_copy(xcat.at[gidx], rows, sem)
        cp.wait()
        gbase = bN + base_local + g * GRP
        pltpu.sync_copy(st_nidx, nidx_o.at[pl.ds(gbase, GRP)])
        pltpu.sync_copy(st_relx, relx_o.at[pl.ds(gbase, GRP)])
        pltpu.sync_copy(st_rely, rely_o.at[pl.ds(gbase, GRP)])
        pltpu.sync_copy(st_relz, relz_o.at[pl.ds(gbase, GRP)])
        pltpu.sync_copy(rows, fx_o.at[pl.ds(gbase * 16, GRP * 16)])
        return 0

    lax.fori_loop(0, NGRP, group_body, 0)


_sc_stage = functools.partial(
    pl.kernel,
    out_type=(
        jax.ShapeDtypeStruct((B * N, NSAMPLE), jnp.int32),
        jax.ShapeDtypeStruct((B * N, NSAMPLE), jnp.float32),
        jax.ShapeDtypeStruct((B * N, NSAMPLE), jnp.float32),
        jax.ShapeDtypeStruct((B * N, NSAMPLE), jnp.float32),
        jax.ShapeDtypeStruct((B * N * NSAMPLE, CF), jnp.float32),
    ),
    mesh=_SC_MESH,
    compiler_params=pltpu.CompilerParams(
        needs_layout_passes=False,
        use_tc_tiling_on_sc=False,
    ),
    scratch_types=[
        pltpu.VMEM((N,), jnp.float32),
        pltpu.VMEM((N,), jnp.float32),
        pltpu.VMEM((N,), jnp.float32),
        pltpu.VMEM((48,), jnp.int32),
        pltpu.VMEM((GRP, 16), jnp.int32),
        pltpu.VMEM((GRP, 16), jnp.float32),
        pltpu.VMEM((GRP, 16), jnp.float32),
        pltpu.VMEM((GRP, 16), jnp.float32),
        pltpu.VMEM((GRP * 16,), jnp.int32),
        pltpu.VMEM((GRP * 16, CF), jnp.float32),
        pltpu.SemaphoreType.DMA,
    ],
)(_sc_body)


RB = 256
NBN = N // RB


def _tc_body(nidx, relx, rely, relz, fx, w_ref, kp_ref, out_ref, skip_ref):
    idx = nidx[...]
    s_iota = lax.broadcasted_iota(jnp.int32, (RB, NSAMPLE), 1)
    pad = jnp.logical_and(idx == idx[:, 0:1], s_iota > 0)
    keep = jnp.where(pad, 0.0, 1.0)
    rx = relx[...]
    ry = rely[...]
    rz = relz[...]
    f3 = fx[:, :, :C_IN]
    acc = jnp.zeros((RB, C_OUT), jnp.float32)
    for k in range(K):
        ax = kp_ref[k, 0]
        ay = kp_ref[k, 1]
        az = kp_ref[k, 2]
        sq = (rx - ax) ** 2 + (ry - ay) ** 2 + (rz - az) ** 2
        awk = jnp.maximum(1.0 - jnp.sqrt(sq + 1e-9) / KP_EXTENT, 0.0)
        wk = jnp.sum(awk[:, :, None] * f3, axis=1)
        acc = acc + jnp.dot(wk, w_ref[k],
                            preferred_element_type=jnp.float32)
    out_ref[0] = acc.T
    xi = fx[:, :, C_IN:] * keep[:, :, None]
    skip_ref[0] = jnp.max(xi, axis=1).T


def _tc_stage(nidx, relx, rely, relz, fx3, weights, kernel_points):
    grid = (B * N // RB,)
    fspec = pl.BlockSpec((RB, NSAMPLE), lambda i: (i, 0))
    return pl.pallas_call(
        _tc_body,
        grid=grid,
        in_specs=[
            pl.BlockSpec((RB, NSAMPLE), lambda i: (i, 0)),
            fspec, fspec, fspec,
            pl.BlockSpec((RB, NSAMPLE, CF), lambda i: (i, 0, 0)),
            pl.BlockSpec((K, C_IN, C_OUT), lambda i: (0, 0, 0)),
            pl.BlockSpec((K, 3), lambda i: (0, 0),
                         memory_space=pltpu.SMEM),
        ],
        out_specs=[
            pl.BlockSpec((1, C_OUT, RB), lambda i: (i // NBN, 0, i % NBN)),
            pl.BlockSpec((1, C_OUT, RB), lambda i: (i // NBN, 0, i % NBN)),
        ],
        out_shape=[
            jax.ShapeDtypeStruct((B, C_OUT, N), jnp.float32),
            jax.ShapeDtypeStruct((B, C_OUT, N), jnp.float32),
        ],
    )(nidx, relx, rely, relz, fx3, weights, kernel_points)


def kernel(p, x, x_in, weights, kernel_points):
    px = p[:, :, 0].reshape(B * N)
    py = p[:, :, 1].reshape(B * N)
    pz = p[:, :, 2].reshape(B * N)
    xcat = jnp.concatenate(
        [jnp.transpose(x, (0, 2, 1)), jnp.transpose(x_in, (0, 2, 1))],
        axis=2).reshape(B * N, CF)
    nidx_f, relx, rely, relz, fx = _sc_stage(px, py, pz, xcat)
    fx3 = fx.reshape(B * N, NSAMPLE, CF)
    out, skip = _tc_stage(nidx_f, relx, rely, relz, fx3,
                          weights, kernel_points)
    return out, p, skip, nidx_f.reshape(B, N, NSAMPLE)


# speedup vs baseline: 21.5169x; 1.0040x over previous
import functools

import jax
import jax.numpy as jnp
from jax import lax
from jax.experimental import pallas as pl
from jax.experimental.pallas import tpu as pltpu
from jax.experimental.pallas import tpu_sc as plsc

B, N, C_IN, C_OUT = 2, 4096, 64, 64
K = 15
RADIUS = 2.5
NSAMPLE = 16
KP_EXTENT = 1.2

NSUB = 16
PTS_PER_SUB = N // NSUB
GRP = 8
NGRP = PTS_PER_SUB // GRP
NCHUNK = N // 16
PAD_XYZ = 1000000.0
SC = NSAMPLE * C_IN


_SC_MESH = plsc.VectorSubcoreMesh(core_axis_name="c", subcore_axis_name="s")


def _sc_body(px, py, pz, xt, xit,
             nidx_o, relx_o, rely_o, relz_o, feat_o, xing_o,
             pxv, pyv, pzv, idxbuf,
             st_nidx, st_relx, st_rely, st_relz,
             gidx, rows_x, rows_xi, sem1, sem2):
    b = lax.axis_index("c")
    w = lax.axis_index("s")
    bN = b * N
    pltpu.sync_copy(px.at[pl.ds(bN, N)], pxv)
    pltpu.sync_copy(py.at[pl.ds(bN, N)], pyv)
    pltpu.sync_copy(pz.at[pl.ds(bN, N)], pzv)
    base_local = w * PTS_PER_SUB
    r2 = jnp.float32(RADIUS * RADIUS)
    lanes = lax.iota(jnp.int32, 16)

    def point_body(t, carry):
        i_loc = base_local + carry + t
        isplat = jnp.full((16,), i_loc, jnp.int32)
        qx = plsc.load_gather(pxv, [isplat])
        qy = plsc.load_gather(pyv, [isplat])
        qz = plsc.load_gather(pzv, [isplat])

        def scan_cond(st):
            j, cnt = st
            return jnp.logical_and(cnt < NSAMPLE, j < NCHUNK)

        def scan_body(st):
            j, cnt = st
            off = j * 16
            jv = lanes + off
            dx = pxv[pl.ds(off, 16)] - qx
            dy = pyv[pl.ds(off, 16)] - qy
            dz = pzv[pl.ds(off, 16)] - qz
            sq = dx * dx + dy * dy + dz * dz
            m = sq <= r2
            plsc.store_compressed(idxbuf.at[pl.ds(cnt, 16)], jv, mask=m)
            return j + 1, cnt + jnp.sum(m.astype(jnp.int32))

        _, cnt = lax.while_loop(scan_cond, scan_body,
                                (jnp.int32(0), jnp.int32(0)))
        f = jnp.minimum(cnt, NSAMPLE)
        idx16 = idxbuf[pl.ds(0, 16)]
        first = idx16[0]
        valid = lanes < f
        idxv = jnp.where(valid, idx16, first)
        gx = plsc.load_gather(pxv, [idxv])
        gy = plsc.load_gather(pyv, [idxv])
        gz = plsc.load_gather(pzv, [idxv])
        big = jnp.float32(PAD_XYZ)
        st_nidx[t, :] = idxv
        st_relx[t, :] = jnp.where(valid, gx - qx, big)
        st_rely[t, :] = jnp.where(valid, gy - qy, big)
        st_relz[t, :] = jnp.where(valid, gz - qz, big)
        gidx[pl.ds(t * 16, 16)] = idxv + bN
        return carry

    def group_body(g, _):
        lax.fori_loop(0, GRP, point_body, g * GRP)
        cp1 = pltpu.async_copy(xt.at[gidx], rows_x, sem1)
        cp2 = pltpu.async_copy(xit.at[gidx], rows_xi, sem2)
        cp1.wait()
        cp2.wait()
        gbase = bN + base_local + g * GRP
        pltpu.sync_copy(st_nidx, nidx_o.at[pl.ds(gbase, GRP)])
        pltpu.sync_copy(st_relx, relx_o.at[pl.ds(gbase, GRP)])
        pltpu.sync_copy(st_rely, rely_o.at[pl.ds(gbase, GRP)])
        pltpu.sync_copy(st_relz, relz_o.at[pl.ds(gbase, GRP)])
        pltpu.sync_copy(rows_x, feat_o.at[pl.ds(gbase * 16, GRP * 16)])
        pltpu.sync_copy(rows_xi, xing_o.at[pl.ds(gbase * 16, GRP * 16)])
        return 0

    lax.fori_loop(0, NGRP, group_body, 0)


_sc_stage = functools.partial(
    pl.kernel,
    out_type=(
        jax.ShapeDtypeStruct((B * N, NSAMPLE), jnp.int32),
        jax.ShapeDtypeStruct((B * N, NSAMPLE), jnp.float32),
        jax.ShapeDtypeStruct((B * N, NSAMPLE), jnp.float32),
        jax.ShapeDtypeStruct((B * N, NSAMPLE), jnp.float32),
        jax.ShapeDtypeStruct((B * N * NSAMPLE, C_IN), jnp.bfloat16),
        jax.ShapeDtypeStruct((B * N * NSAMPLE, C_IN), jnp.bfloat16),
    ),
    mesh=_SC_MESH,
    compiler_params=pltpu.CompilerParams(
        needs_layout_passes=False,
        use_tc_tiling_on_sc=False,
    ),
    scratch_types=[
        pltpu.VMEM((N,), jnp.float32),
        pltpu.VMEM((N,), jnp.float32),
        pltpu.VMEM((N,), jnp.float32),
        pltpu.VMEM((48,), jnp.int32),
        pltpu.VMEM((GRP, 16), jnp.int32),
        pltpu.VMEM((GRP, 16), jnp.float32),
        pltpu.VMEM((GRP, 16), jnp.float32),
        pltpu.VMEM((GRP, 16), jnp.float32),
        pltpu.VMEM((GRP * 16,), jnp.int32),
        pltpu.VMEM((GRP * 16, C_IN), jnp.bfloat16),
        pltpu.VMEM((GRP * 16, C_IN), jnp.bfloat16),
        pltpu.SemaphoreType.DMA,
        pltpu.SemaphoreType.DMA,
    ],
)(_sc_body)


RB = 256
NBN = N // RB


def _tc_body(nidx, relx, rely, relz, f2, xi2, wrep_ref, kp_ref,
             out_ref, skip_ref):
    idx = nidx[...]
    s_iota = lax.broadcasted_iota(jnp.int32, (RB, NSAMPLE), 1)
    pad = jnp.logical_and(idx == idx[:, 0:1], s_iota > 0)
    keep = jnp.where(pad, 0.0, 1.0)
    rx = relx[...]
    ry = rely[...]
    rz = relz[...]
    rep_r = lax.broadcasted_iota(jnp.int32, (NSAMPLE, SC), 0)
    rep_c = lax.broadcasted_iota(jnp.int32, (NSAMPLE, SC), 1)
    rep16 = jnp.where(rep_c // C_IN == rep_r, 1.0, 0.0).astype(jnp.bfloat16)
    f2v = f2[...]
    acc = jnp.zeros((RB, C_OUT), jnp.float32)
    for k in range(K):
        ax = kp_ref[k, 0]
        ay = kp_ref[k, 1]
        az = kp_ref[k, 2]
        sq = (rx - ax) ** 2 + (ry - ay) ** 2 + (rz - az) ** 2
        awk = jnp.maximum(1.0 - jnp.sqrt(sq + 1e-9) / KP_EXTENT, 0.0)
        awb = jnp.dot(awk.astype(jnp.bfloat16), rep16,
                      preferred_element_type=jnp.float32)
        acc = acc + jnp.dot(awb.astype(jnp.bfloat16) * f2v, wrep_ref[k],
                            preferred_element_type=jnp.float32)
    out_ref[0] = acc.T
    keepb = jnp.dot(keep.astype(jnp.bfloat16), rep16,
                    preferred_element_type=jnp.float32)
    xim = xi2[...] * keepb.astype(jnp.bfloat16)
    m = xim[:, 0:C_IN]
    for s in range(1, NSAMPLE):
        m = jnp.maximum(m, xim[:, s * C_IN:(s + 1) * C_IN])
    skip_ref[0] = m.astype(jnp.float32).T


def _tc_stage(nidx, relx, rely, relz, f2, xi2, wrep, kernel_points):
    grid = (B * N // RB,)
    fspec = pl.BlockSpec((RB, NSAMPLE), lambda i: (i, 0))
    return pl.pallas_call(
        _tc_body,
        grid=grid,
        in_specs=[
            pl.BlockSpec((RB, NSAMPLE), lambda i: (i, 0)),
            fspec, fspec, fspec,
            pl.BlockSpec((RB, SC), lambda i: (i, 0)),
            pl.BlockSpec((RB, SC), lambda i: (i, 0)),
            pl.BlockSpec((K, SC, C_OUT), lambda i: (0, 0, 0)),
            pl.BlockSpec((K, 3), lambda i: (0, 0),
                         memory_space=pltpu.SMEM),
        ],
        out_specs=[
            pl.BlockSpec((1, C_OUT, RB), lambda i: (i // NBN, 0, i % NBN)),
            pl.BlockSpec((1, C_OUT, RB), lambda i: (i // NBN, 0, i % NBN)),
        ],
        out_shape=[
            jax.ShapeDtypeStruct((B, C_OUT, N), jnp.float32),
            jax.ShapeDtypeStruct((B, C_OUT, N), jnp.float32),
        ],
    )(nidx, relx, rely, relz, f2, xi2, wrep, kernel_points)


def kernel(p, x, x_in, weights, kernel_points):
    px = p[:, :, 0].reshape(B * N)
    py = p[:, :, 1].reshape(B * N)
    pz = p[:, :, 2].reshape(B * N)
    xt = jnp.transpose(x, (0, 2, 1)).reshape(B * N, C_IN).astype(jnp.bfloat16)
    xit = jnp.transpose(x_in, (0, 2, 1)).reshape(B * N, C_IN).astype(jnp.bfloat16)
    nidx_f, relx, rely, relz, feat, xing = _sc_stage(px, py, pz, xt, xit)
    f2 = feat.reshape(B * N, SC)
    xi2 = xing.reshape(B * N, SC)
    wrep = jnp.tile(weights, (1, NSAMPLE, 1)).astype(jnp.bfloat16)
    out, skip = _tc_stage(nidx_f, relx, rely, relz, f2, xi2,
                          wrep, kernel_points)
    return out, p, skip, nidx_f.reshape(B, N, NSAMPLE)


# speedup vs baseline: 21.5476x; 1.0014x over previous
import functools

import jax
import jax.numpy as jnp
from jax import lax
from jax.experimental import pallas as pl
from jax.experimental.pallas import tpu as pltpu
from jax.experimental.pallas import tpu_sc as plsc

B, N, C_IN, C_OUT = 2, 4096, 64, 64
K = 15
RADIUS = 2.5
NSAMPLE = 16
KP_EXTENT = 1.2

NSUB = 16
PTS_PER_SUB = N // NSUB
GRP = 8
NGRP = PTS_PER_SUB // GRP
NCHUNK = N // 16
PAD_XYZ = 1000000.0
SC = NSAMPLE * C_IN


_SC_MESH = plsc.VectorSubcoreMesh(core_axis_name="c", subcore_axis_name="s")


def _sc_body(px, py, pz, xt, xit,
             nidx_o, relx_o, rely_o, relz_o, feat_o, xing_o,
             pxv, pyv, pzv, idxbuf,
             st_nidx, st_relx, st_rely, st_relz,
             gidx, rows_x, rows_xi, sem1, sem2):
    b = lax.axis_index("c")
    w = lax.axis_index("s")
    bN = b * N
    with jax.named_scope("stage_coords"):
        pltpu.sync_copy(px.at[pl.ds(bN, N)], pxv)
        pltpu.sync_copy(py.at[pl.ds(bN, N)], pyv)
        pltpu.sync_copy(pz.at[pl.ds(bN, N)], pzv)
    base_local = w * PTS_PER_SUB
    r2 = jnp.float32(RADIUS * RADIUS)
    lanes = lax.iota(jnp.int32, 16)

    def point_body(t, carry):
        i_loc = base_local + carry + t
        isplat = jnp.full((16,), i_loc, jnp.int32)
        qx = plsc.load_gather(pxv, [isplat])
        qy = plsc.load_gather(pyv, [isplat])
        qz = plsc.load_gather(pzv, [isplat])

        def scan_cond(st):
            j, cnt = st
            return jnp.logical_and(cnt < NSAMPLE, j < NCHUNK)

        def scan_body(st):
            j, cnt = st
            off = j * 16
            jv = lanes + off
            dx = pxv[pl.ds(off, 16)] - qx
            dy = pyv[pl.ds(off, 16)] - qy
            dz = pzv[pl.ds(off, 16)] - qz
            sq = dx * dx + dy * dy + dz * dz
            m = sq <= r2
            plsc.store_compressed(idxbuf.at[pl.ds(cnt, 16)], jv, mask=m)
            return j + 1, cnt + jnp.sum(m.astype(jnp.int32))

        _, cnt = lax.while_loop(scan_cond, scan_body,
                                (jnp.int32(0), jnp.int32(0)))
        f = jnp.minimum(cnt, NSAMPLE)
        idx16 = idxbuf[pl.ds(0, 16)]
        first = idx16[0]
        valid = lanes < f
        idxv = jnp.where(valid, idx16, first)
        gx = plsc.load_gather(pxv, [idxv])
        gy = plsc.load_gather(pyv, [idxv])
        gz = plsc.load_gather(pzv, [idxv])
        big = jnp.float32(PAD_XYZ)
        st_nidx[t, :] = idxv
        st_relx[t, :] = jnp.where(valid, gx - qx, big)
        st_rely[t, :] = jnp.where(valid, gy - qy, big)
        st_relz[t, :] = jnp.where(valid, gz - qz, big)
        gidx[pl.ds(t * 16, 16)] = idxv + bN
        return carry

    def group_body(g, _):
        with jax.named_scope("ballq"):
            lax.fori_loop(0, GRP, point_body, g * GRP)
        with jax.named_scope("gather_dma"):
            cp1 = pltpu.async_copy(xt.at[gidx], rows_x, sem1)
            cp2 = pltpu.async_copy(xit.at[gidx], rows_xi, sem2)
            cp1.wait()
            cp2.wait()
        gbase = bN + base_local + g * GRP
        with jax.named_scope("copyout"):
            pltpu.sync_copy(st_nidx, nidx_o.at[pl.ds(gbase, GRP)])
            pltpu.sync_copy(st_relx, relx_o.at[pl.ds(gbase, GRP)])
            pltpu.sync_copy(st_rely, rely_o.at[pl.ds(gbase, GRP)])
            pltpu.sync_copy(st_relz, relz_o.at[pl.ds(gbase, GRP)])
            pltpu.sync_copy(rows_x, feat_o.at[pl.ds(gbase * 16, GRP * 16)])
            pltpu.sync_copy(rows_xi, xing_o.at[pl.ds(gbase * 16, GRP * 16)])
        return 0

    lax.fori_loop(0, NGRP, group_body, 0)


_sc_stage = functools.partial(
    pl.kernel,
    out_type=(
        jax.ShapeDtypeStruct((B * N, NSAMPLE), jnp.int32),
        jax.ShapeDtypeStruct((B * N, NSAMPLE), jnp.float32),
        jax.ShapeDtypeStruct((B * N, NSAMPLE), jnp.float32),
        jax.ShapeDtypeStruct((B * N, NSAMPLE), jnp.float32),
        jax.ShapeDtypeStruct((B * N * NSAMPLE, C_IN), jnp.bfloat16),
        jax.ShapeDtypeStruct((B * N * NSAMPLE, C_IN), jnp.bfloat16),
    ),
    mesh=_SC_MESH,
    compiler_params=pltpu.CompilerParams(
        needs_layout_passes=False,
        use_tc_tiling_on_sc=False,
    ),
    scratch_types=[
        pltpu.VMEM((N,), jnp.float32),
        pltpu.VMEM((N,), jnp.float32),
        pltpu.VMEM((N,), jnp.float32),
        pltpu.VMEM((48,), jnp.int32),
        pltpu.VMEM((GRP, 16), jnp.int32),
        pltpu.VMEM((GRP, 16), jnp.float32),
        pltpu.VMEM((GRP, 16), jnp.float32),
        pltpu.VMEM((GRP, 16), jnp.float32),
        pltpu.VMEM((GRP * 16,), jnp.int32),
        pltpu.VMEM((GRP * 16, C_IN), jnp.bfloat16),
        pltpu.VMEM((GRP * 16, C_IN), jnp.bfloat16),
        pltpu.SemaphoreType.DMA,
        pltpu.SemaphoreType.DMA,
    ],
)(_sc_body)


RB = 256
NBN = N // RB


def _tc_body(nidx, relx, rely, relz, f2, xi2, wrep_ref, kp_ref,
             out_ref, skip_ref):
    idx = nidx[...]
    s_iota = lax.broadcasted_iota(jnp.int32, (RB, NSAMPLE), 1)
    pad = jnp.logical_and(idx == idx[:, 0:1], s_iota > 0)
    keep = jnp.where(pad, 0.0, 1.0)
    rx = relx[...]
    ry = rely[...]
    rz = relz[...]
    rep_r = lax.broadcasted_iota(jnp.int32, (NSAMPLE, SC), 0)
    rep_c = lax.broadcasted_iota(jnp.int32, (NSAMPLE, SC), 1)
    rep16 = jnp.where(rep_c // C_IN == rep_r, 1.0, 0.0).astype(jnp.bfloat16)
    f2v = f2[...]
    acc = jnp.zeros((RB, C_OUT), jnp.float32)
    for k in range(K):
        ax = kp_ref[k, 0]
        ay = kp_ref[k, 1]
        az = kp_ref[k, 2]
        sq = (rx - ax) ** 2 + (ry - ay) ** 2 + (rz - az) ** 2
        awk = jnp.maximum(1.0 - jnp.sqrt(sq + 1e-9) / KP_EXTENT, 0.0)
        awb = jnp.dot(awk.astype(jnp.bfloat16), rep16,
                      preferred_element_type=jnp.float32)
        acc = acc + jnp.dot(awb.astype(jnp.bfloat16) * f2v, wrep_ref[k],
                            preferred_element_type=jnp.float32)
    out_ref[0] = acc.T
    keepb = jnp.dot(keep.astype(jnp.bfloat16), rep16,
                    preferred_element_type=jnp.float32)
    xim = xi2[...] * keepb.astype(jnp.bfloat16)
    m = xim[:, 0:C_IN]
    for s in range(1, NSAMPLE):
        m = jnp.maximum(m, xim[:, s * C_IN:(s + 1) * C_IN])
    skip_ref[0] = m.astype(jnp.float32).T


def _tc_stage(nidx, relx, rely, relz, f2, xi2, wrep, kernel_points):
    grid = (B * N // RB,)
    fspec = pl.BlockSpec((RB, NSAMPLE), lambda i: (i, 0))
    return pl.pallas_call(
        _tc_body,
        grid=grid,
        in_specs=[
            pl.BlockSpec((RB, NSAMPLE), lambda i: (i, 0)),
            fspec, fspec, fspec,
            pl.BlockSpec((RB, SC), lambda i: (i, 0)),
            pl.BlockSpec((RB, SC), lambda i: (i, 0)),
            pl.BlockSpec((K, SC, C_OUT), lambda i: (0, 0, 0)),
            pl.BlockSpec((K, 3), lambda i: (0, 0),
                         memory_space=pltpu.SMEM),
        ],
        out_specs=[
            pl.BlockSpec((1, C_OUT, RB), lambda i: (i // NBN, 0, i % NBN)),
            pl.BlockSpec((1, C_OUT, RB), lambda i: (i // NBN, 0, i % NBN)),
        ],
        out_shape=[
            jax.ShapeDtypeStruct((B, C_OUT, N), jnp.float32),
            jax.ShapeDtypeStruct((B, C_OUT, N), jnp.float32),
        ],
    )(nidx, relx, rely, relz, f2, xi2, wrep, kernel_points)


def kernel(p, x, x_in, weights, kernel_points):
    px = p[:, :, 0].reshape(B * N)
    py = p[:, :, 1].reshape(B * N)
    pz = p[:, :, 2].reshape(B * N)
    xt = jnp.transpose(x, (0, 2, 1)).reshape(B * N, C_IN).astype(jnp.bfloat16)
    xit = jnp.transpose(x_in, (0, 2, 1)).reshape(B * N, C_IN).astype(jnp.bfloat16)
    nidx_f, relx, rely, relz, feat, xing = _sc_stage(px, py, pz, xt, xit)
    f2 = feat.reshape(B * N, SC)
    xi2 = xing.reshape(B * N, SC)
    wrep = jnp.tile(weights, (1, NSAMPLE, 1)).astype(jnp.bfloat16)
    out, skip = _tc_stage(nidx_f, relx, rely, relz, f2, xi2,
                          wrep, kernel_points)
    return out, p, skip, nidx_f.reshape(B, N, NSAMPLE)


# speedup vs baseline: 21.5917x; 1.0020x over previous
import functools

import jax
import jax.numpy as jnp
from jax import lax
from jax.experimental import pallas as pl
from jax.experimental.pallas import tpu as pltpu
from jax.experimental.pallas import tpu_sc as plsc

B, N, C_IN, C_OUT = 2, 4096, 64, 64
K = 15
RADIUS = 2.5
NSAMPLE = 16
KP_EXTENT = 1.2

NSUB = 16
PTS_PER_SUB = N // NSUB
GRP = 8
NGRP = PTS_PER_SUB // GRP
NCHUNK = N // 16
PAD_XYZ = 1000000.0
SC = NSAMPLE * C_IN


_SC_MESH = plsc.VectorSubcoreMesh(core_axis_name="c", subcore_axis_name="s")


def _sc_body(px, py, pz, xt, xit,
             nidx_o, relx_o, rely_o, relz_o, feat_o, xing_o,
             pxv, pyv, pzv, idxbuf,
             st_nidx, st_relx, st_rely, st_relz,
             gidx, rows_x, rows_xi, sem1, sem2):
    b = lax.axis_index("c")
    w = lax.axis_index("s")
    bN = b * N
    pltpu.sync_copy(px.at[pl.ds(bN, N)], pxv)
    pltpu.sync_copy(py.at[pl.ds(bN, N)], pyv)
    pltpu.sync_copy(pz.at[pl.ds(bN, N)], pzv)
    base_local = w * PTS_PER_SUB
    r2 = jnp.float32(RADIUS * RADIUS)
    lanes = lax.iota(jnp.int32, 16)

    def point_body(t, carry):
        i_loc = base_local + carry + t
        isplat = jnp.full((16,), i_loc, jnp.int32)
        qx = plsc.load_gather(pxv, [isplat])
        qy = plsc.load_gather(pyv, [isplat])
        qz = plsc.load_gather(pzv, [isplat])

        def scan_cond(st):
            j, cnt = st
            return jnp.logical_and(cnt < NSAMPLE, j < NCHUNK)

        def scan_body(st):
            j, cnt = st
            off = j * 16
            jv = lanes + off
            dx = pxv[pl.ds(off, 16)] - qx
            dy = pyv[pl.ds(off, 16)] - qy
            dz = pzv[pl.ds(off, 16)] - qz
            sq = dx * dx + dy * dy + dz * dz
            m = sq <= r2
            plsc.store_compressed(idxbuf.at[pl.ds(cnt, 16)], jv, mask=m)
            cnt1 = cnt + jnp.sum(m.astype(jnp.int32))
            off2 = off + 16
            jv2 = lanes + off2
            dx2 = pxv[pl.ds(off2, 16)] - qx
            dy2 = pyv[pl.ds(off2, 16)] - qy
            dz2 = pzv[pl.ds(off2, 16)] - qz
            sq2 = dx2 * dx2 + dy2 * dy2 + dz2 * dz2
            m2 = sq2 <= r2
            plsc.store_compressed(idxbuf.at[pl.ds(cnt1, 16)], jv2, mask=m2)
            return j + 2, cnt1 + jnp.sum(m2.astype(jnp.int32))

        _, cnt = lax.while_loop(scan_cond, scan_body,
                                (jnp.int32(0), jnp.int32(0)))
        f = jnp.minimum(cnt, NSAMPLE)
        idx16 = idxbuf[pl.ds(0, 16)]
        first = idx16[0]
        valid = lanes < f
        idxv = jnp.where(valid, idx16, first)
        gx = plsc.load_gather(pxv, [idxv])
        gy = plsc.load_gather(pyv, [idxv])
        gz = plsc.load_gather(pzv, [idxv])
        big = jnp.float32(PAD_XYZ)
        st_nidx[t, :] = idxv
        st_relx[t, :] = jnp.where(valid, gx - qx, big)
        st_rely[t, :] = jnp.where(valid, gy - qy, big)
        st_relz[t, :] = jnp.where(valid, gz - qz, big)
        gidx[pl.ds(t * 16, 16)] = idxv + bN
        return carry

    def group_body(g, _):
        lax.fori_loop(0, GRP, point_body, g * GRP)
        cp1 = pltpu.async_copy(xt.at[gidx], rows_x, sem1)
        cp2 = pltpu.async_copy(xit.at[gidx], rows_xi, sem2)
        cp1.wait()
        cp2.wait()
        gbase = bN + base_local + g * GRP
        pltpu.sync_copy(st_nidx, nidx_o.at[pl.ds(gbase, GRP)])
        pltpu.sync_copy(st_relx, relx_o.at[pl.ds(gbase, GRP)])
        pltpu.sync_copy(st_rely, rely_o.at[pl.ds(gbase, GRP)])
        pltpu.sync_copy(st_relz, relz_o.at[pl.ds(gbase, GRP)])
        pltpu.sync_copy(rows_x, feat_o.at[pl.ds(gbase * 16, GRP * 16)])
        pltpu.sync_copy(rows_xi, xing_o.at[pl.ds(gbase * 16, GRP * 16)])
        return 0

    lax.fori_loop(0, NGRP, group_body, 0)


_sc_stage = functools.partial(
    pl.kernel,
    out_type=(
        jax.ShapeDtypeStruct((B * N, NSAMPLE), jnp.int32),
        jax.ShapeDtypeStruct((B * N, NSAMPLE), jnp.float32),
        jax.ShapeDtypeStruct((B * N, NSAMPLE), jnp.float32),
        jax.ShapeDtypeStruct((B * N, NSAMPLE), jnp.float32),
        jax.ShapeDtypeStruct((B * N * NSAMPLE, C_IN), jnp.bfloat16),
        jax.ShapeDtypeStruct((B * N * NSAMPLE, C_IN), jnp.bfloat16),
    ),
    mesh=_SC_MESH,
    compiler_params=pltpu.CompilerParams(
        needs_layout_passes=False,
        use_tc_tiling_on_sc=False,
    ),
    scratch_types=[
        pltpu.VMEM((N,), jnp.float32),
        pltpu.VMEM((N,), jnp.float32),
        pltpu.VMEM((N,), jnp.float32),
        pltpu.VMEM((48,), jnp.int32),
        pltpu.VMEM((GRP, 16), jnp.int32),
        pltpu.VMEM((GRP, 16), jnp.float32),
        pltpu.VMEM((GRP, 16), jnp.float32),
        pltpu.VMEM((GRP, 16), jnp.float32),
        pltpu.VMEM((GRP * 16,), jnp.int32),
        pltpu.VMEM((GRP * 16, C_IN), jnp.bfloat16),
        pltpu.VMEM((GRP * 16, C_IN), jnp.bfloat16),
        pltpu.SemaphoreType.DMA,
        pltpu.SemaphoreType.DMA,
    ],
)(_sc_body)


RB = 256
NBN = N // RB


def _tc_body(nidx, relx, rely, relz, f2, xi2, wrep_ref, kp_ref,
             out_ref, skip_ref):
    idx = nidx[...]
    s_iota = lax.broadcasted_iota(jnp.int32, (RB, NSAMPLE), 1)
    pad = jnp.logical_and(idx == idx[:, 0:1], s_iota > 0)
    keep = jnp.where(pad, 0.0, 1.0)
    rx = relx[...]
    ry = rely[...]
    rz = relz[...]
    rep_r = lax.broadcasted_iota(jnp.int32, (NSAMPLE, SC), 0)
    rep_c = lax.broadcasted_iota(jnp.int32, (NSAMPLE, SC), 1)
    rep16 = jnp.where(rep_c // C_IN == rep_r, 1.0, 0.0).astype(jnp.bfloat16)
    f2v = f2[...]
    acc = jnp.zeros((RB, C_OUT), jnp.float32)
    for k in range(K):
        ax = kp_ref[k, 0]
        ay = kp_ref[k, 1]
        az = kp_ref[k, 2]
        sq = (rx - ax) ** 2 + (ry - ay) ** 2 + (rz - az) ** 2
        awk = jnp.maximum(1.0 - jnp.sqrt(sq + 1e-9) / KP_EXTENT, 0.0)
        awb = jnp.dot(awk.astype(jnp.bfloat16), rep16,
                      preferred_element_type=jnp.float32)
        acc = acc + jnp.dot(awb.astype(jnp.bfloat16) * f2v, wrep_ref[k],
                            preferred_element_type=jnp.float32)
    out_ref[0] = acc.T
    keepb = jnp.dot(keep.astype(jnp.bfloat16), rep16,
                    preferred_element_type=jnp.float32)
    xim = xi2[...] * keepb.astype(jnp.bfloat16)
    m = xim[:, 0:C_IN]
    for s in range(1, NSAMPLE):
        m = jnp.maximum(m, xim[:, s * C_IN:(s + 1) * C_IN])
    skip_ref[0] = m.astype(jnp.float32).T


def _tc_stage(nidx, relx, rely, relz, f2, xi2, wrep, kernel_points):
    grid = (B * N // RB,)
    fspec = pl.BlockSpec((RB, NSAMPLE), lambda i: (i, 0))
    return pl.pallas_call(
        _tc_body,
        grid=grid,
        in_specs=[
            pl.BlockSpec((RB, NSAMPLE), lambda i: (i, 0)),
            fspec, fspec, fspec,
            pl.BlockSpec((RB, SC), lambda i: (i, 0)),
            pl.BlockSpec((RB, SC), lambda i: (i, 0)),
            pl.BlockSpec((K, SC, C_OUT), lambda i: (0, 0, 0)),
            pl.BlockSpec((K, 3), lambda i: (0, 0),
                         memory_space=pltpu.SMEM),
        ],
        out_specs=[
            pl.BlockSpec((1, C_OUT, RB), lambda i: (i // NBN, 0, i % NBN)),
            pl.BlockSpec((1, C_OUT, RB), lambda i: (i // NBN, 0, i % NBN)),
        ],
        out_shape=[
            jax.ShapeDtypeStruct((B, C_OUT, N), jnp.float32),
            jax.ShapeDtypeStruct((B, C_OUT, N), jnp.float32),
        ],
    )(nidx, relx, rely, relz, f2, xi2, wrep, kernel_points)


def kernel(p, x, x_in, weights, kernel_points):
    px = p[:, :, 0].reshape(B * N)
    py = p[:, :, 1].reshape(B * N)
    pz = p[:, :, 2].reshape(B * N)
    xt = jnp.transpose(x, (0, 2, 1)).reshape(B * N, C_IN).astype(jnp.bfloat16)
    xit = jnp.transpose(x_in, (0, 2, 1)).reshape(B * N, C_IN).astype(jnp.bfloat16)
    nidx_f, relx, rely, relz, feat, xing = _sc_stage(px, py, pz, xt, xit)
    f2 = feat.reshape(B * N, SC)
    xi2 = xing.reshape(B * N, SC)
    wrep = jnp.tile(weights, (1, NSAMPLE, 1)).astype(jnp.bfloat16)
    out, skip = _tc_stage(nidx_f, relx, rely, relz, f2, xi2,
                          wrep, kernel_points)
    return out, p, skip, nidx_f.reshape(B, N, NSAMPLE)


# speedup vs baseline: 21.9661x; 1.0173x over previous
import functools

import jax
import jax.numpy as jnp
from jax import lax
from jax.experimental import pallas as pl
from jax.experimental.pallas import tpu as pltpu
from jax.experimental.pallas import tpu_sc as plsc

B, N, C_IN, C_OUT = 2, 4096, 64, 64
K = 15
RADIUS = 2.5
NSAMPLE = 16
KP_EXTENT = 1.2

NSUB = 16
PTS_PER_SUB = N // NSUB
GRP = 8
NGRP = PTS_PER_SUB // GRP
NCHUNK = N // 16
PAD_XYZ = 1000000.0
SC = NSAMPLE * C_IN


_SC_MESH = plsc.VectorSubcoreMesh(core_axis_name="c", subcore_axis_name="s")


def _sc_body(px, py, pz, xt, xit,
             nidx_o, relx_o, rely_o, relz_o, feat_o, xing_o,
             pxv, pyv, pzv, idxbuf,
             st_nidx, st_relx, st_rely, st_relz,
             gidx0, gidx1, rows_x0, rows_xi0, rows_x1, rows_xi1,
             gsx0, gsxi0, gsx1, gsxi1, wsx0, wsxi0, wsx1, wsxi1):
    b = lax.axis_index("c")
    w = lax.axis_index("s")
    bN = b * N
    pltpu.sync_copy(px.at[pl.ds(bN, N)], pxv)
    pltpu.sync_copy(py.at[pl.ds(bN, N)], pyv)
    pltpu.sync_copy(pz.at[pl.ds(bN, N)], pzv)
    base_local = w * PTS_PER_SUB
    r2 = jnp.float32(RADIUS * RADIUS)
    lanes = lax.iota(jnp.int32, 16)

    def scan_group(g, gidx):
        def point_body(t, carry):
            i_loc = base_local + carry + t
            isplat = jnp.full((16,), i_loc, jnp.int32)
            qx = plsc.load_gather(pxv, [isplat])
            qy = plsc.load_gather(pyv, [isplat])
            qz = plsc.load_gather(pzv, [isplat])

            def scan_cond(st):
                j, cnt = st
                return jnp.logical_and(cnt < NSAMPLE, j < NCHUNK)

            def scan_body(st):
                j, cnt = st
                off = j * 16
                jv = lanes + off
                dx = pxv[pl.ds(off, 16)] - qx
                dy = pyv[pl.ds(off, 16)] - qy
                dz = pzv[pl.ds(off, 16)] - qz
                sq = dx * dx + dy * dy + dz * dz
                m = sq <= r2
                plsc.store_compressed(idxbuf.at[pl.ds(cnt, 16)], jv, mask=m)
                return j + 1, cnt + jnp.sum(m.astype(jnp.int32))

            _, cnt = lax.while_loop(scan_cond, scan_body,
                                    (jnp.int32(0), jnp.int32(0)))
            f = jnp.minimum(cnt, NSAMPLE)
            idx16 = idxbuf[pl.ds(0, 16)]
            first = idx16[0]
            valid = lanes < f
            idxv = jnp.where(valid, idx16, first)
            gx = plsc.load_gather(pxv, [idxv])
            gy = plsc.load_gather(pyv, [idxv])
            gz = plsc.load_gather(pzv, [idxv])
            big = jnp.float32(PAD_XYZ)
            st_nidx[t, :] = idxv
            st_relx[t, :] = jnp.where(valid, gx - qx, big)
            st_rely[t, :] = jnp.where(valid, gy - qy, big)
            st_relz[t, :] = jnp.where(valid, gz - qz, big)
            gidx[pl.ds(t * 16, 16)] = idxv + bN
            return carry

        lax.fori_loop(0, GRP, point_body, g * GRP)

    def emit_st(g):
        gbase = bN + base_local + g * GRP
        pltpu.sync_copy(st_nidx, nidx_o.at[pl.ds(gbase, GRP)])
        pltpu.sync_copy(st_relx, relx_o.at[pl.ds(gbase, GRP)])
        pltpu.sync_copy(st_rely, rely_o.at[pl.ds(gbase, GRP)])
        pltpu.sync_copy(st_relz, relz_o.at[pl.ds(gbase, GRP)])

    def fslice(g):
        gbase = bN + base_local + g * GRP
        return pl.ds(gbase * 16, GRP * 16)

    def issue_gather(gidx, rx, rxi, sx, sxi):
        pltpu.async_copy(xt.at[gidx], rx, sx)
        pltpu.async_copy(xit.at[gidx], rxi, sxi)

    def wait_gather(gidx, rx, rxi, sx, sxi):
        pltpu.make_async_copy(xt.at[gidx], rx, sx).wait()
        pltpu.make_async_copy(xit.at[gidx], rxi, sxi).wait()

    def issue_wb(g, rx, rxi, sx, sxi):
        pltpu.async_copy(rx, feat_o.at[fslice(g)], sx)
        pltpu.async_copy(rxi, xing_o.at[fslice(g)], sxi)

    def wait_wb(g, rx, rxi, sx, sxi):
        pltpu.make_async_copy(rx, feat_o.at[fslice(g)], sx).wait()
        pltpu.make_async_copy(rxi, xing_o.at[fslice(g)], sxi).wait()

    scan_group(0, gidx0)
    issue_gather(gidx0, rows_x0, rows_xi0, gsx0, gsxi0)
    emit_st(0)
    scan_group(1, gidx1)
    issue_gather(gidx1, rows_x1, rows_xi1, gsx1, gsxi1)
    emit_st(1)
    wait_gather(gidx0, rows_x0, rows_xi0, gsx0, gsxi0)
    issue_wb(0, rows_x0, rows_xi0, wsx0, wsxi0)

    def pair_body(h, _):
        g0 = 2 * h
        g1 = g0 + 1
        scan_group(g0, gidx0)
        wait_wb(g0 - 2, rows_x0, rows_xi0, wsx0, wsxi0)
        issue_gather(gidx0, rows_x0, rows_xi0, gsx0, gsxi0)
        emit_st(g0)
        wait_gather(gidx1, rows_x1, rows_xi1, gsx1, gsxi1)
        issue_wb(g0 - 1, rows_x1, rows_xi1, wsx1, wsxi1)
        scan_group(g1, gidx1)
        wait_wb(g0 - 1, rows_x1, rows_xi1, wsx1, wsxi1)
        issue_gather(gidx1, rows_x1, rows_xi1, gsx1, gsxi1)
        emit_st(g1)
        wait_gather(gidx0, rows_x0, rows_xi0, gsx0, gsxi0)
        issue_wb(g0, rows_x0, rows_xi0, wsx0, wsxi0)
        return 0

    lax.fori_loop(1, NGRP // 2, pair_body, 0)
    wait_gather(gidx1, rows_x1, rows_xi1, gsx1, gsxi1)
    pltpu.sync_copy(rows_x1, feat_o.at[fslice(NGRP - 1)])
    pltpu.sync_copy(rows_xi1, xing_o.at[fslice(NGRP - 1)])
    wait_wb(NGRP - 2, rows_x0, rows_xi0, wsx0, wsxi0)


_sc_stage = functools.partial(
    pl.kernel,
    out_type=(
        jax.ShapeDtypeStruct((B * N, NSAMPLE), jnp.int32),
        jax.ShapeDtypeStruct((B * N, NSAMPLE), jnp.float32),
        jax.ShapeDtypeStruct((B * N, NSAMPLE), jnp.float32),
        jax.ShapeDtypeStruct((B * N, NSAMPLE), jnp.float32),
        jax.ShapeDtypeStruct((B * N * NSAMPLE, C_IN), jnp.bfloat16),
        jax.ShapeDtypeStruct((B * N * NSAMPLE, C_IN), jnp.bfloat16),
    ),
    mesh=_SC_MESH,
    compiler_params=pltpu.CompilerParams(
        needs_layout_passes=False,
        use_tc_tiling_on_sc=False,
    ),
    scratch_types=[
        pltpu.VMEM((N,), jnp.float32),
        pltpu.VMEM((N,), jnp.float32),
        pltpu.VMEM((N,), jnp.float32),
        pltpu.VMEM((48,), jnp.int32),
        pltpu.VMEM((GRP, 16), jnp.int32),
        pltpu.VMEM((GRP, 16), jnp.float32),
        pltpu.VMEM((GRP, 16), jnp.float32),
        pltpu.VMEM((GRP, 16), jnp.float32),
        pltpu.VMEM((GRP * 16,), jnp.int32),
        pltpu.VMEM((GRP * 16,), jnp.int32),
        pltpu.VMEM((GRP * 16, C_IN), jnp.bfloat16),
        pltpu.VMEM((GRP * 16, C_IN), jnp.bfloat16),
        pltpu.VMEM((GRP * 16, C_IN), jnp.bfloat16),
        pltpu.VMEM((GRP * 16, C_IN), jnp.bfloat16),
        pltpu.SemaphoreType.DMA,
        pltpu.SemaphoreType.DMA,
        pltpu.SemaphoreType.DMA,
        pltpu.SemaphoreType.DMA,
        pltpu.SemaphoreType.DMA,
        pltpu.SemaphoreType.DMA,
        pltpu.SemaphoreType.DMA,
        pltpu.SemaphoreType.DMA,
    ],
)(_sc_body)


RB = 256
NBN = N // RB


def _tc_body(nidx, relx, rely, relz, f2, xi2, wrep_ref, kp_ref,
             out_ref, skip_ref):
    idx = nidx[...]
    s_iota = lax.broadcasted_iota(jnp.int32, (RB, NSAMPLE), 1)
    pad = jnp.logical_and(idx == idx[:, 0:1], s_iota > 0)
    keep = jnp.where(pad, 0.0, 1.0)
    rx = relx[...]
    ry = rely[...]
    rz = relz[...]
    rep_r = lax.broadcasted_iota(jnp.int32, (NSAMPLE, SC), 0)
    rep_c = lax.broadcasted_iota(jnp.int32, (NSAMPLE, SC), 1)
    rep16 = jnp.where(rep_c // C_IN == rep_r, 1.0, 0.0).astype(jnp.bfloat16)
    f2v = f2[...]
    acc = jnp.zeros((RB, C_OUT), jnp.float32)
    for k in range(K):
        ax = kp_ref[k, 0]
        ay = kp_ref[k, 1]
        az = kp_ref[k, 2]
        sq = (rx - ax) ** 2 + (ry - ay) ** 2 + (rz - az) ** 2
        awk = jnp.maximum(1.0 - jnp.sqrt(sq + 1e-9) / KP_EXTENT, 0.0)
        awb = jnp.dot(awk.astype(jnp.bfloat16), rep16,
                      preferred_element_type=jnp.float32)
        acc = acc + jnp.dot(awb.astype(jnp.bfloat16) * f2v, wrep_ref[k],
                            preferred_element_type=jnp.float32)
    out_ref[0] = acc.T
    keepb = jnp.dot(keep.astype(jnp.bfloat16), rep16,
                    preferred_element_type=jnp.float32)
    xim = xi2[...] * keepb.astype(jnp.bfloat16)
    m = xim[:, 0:C_IN]
    for s in range(1, NSAMPLE):
        m = jnp.maximum(m, xim[:, s * C_IN:(s + 1) * C_IN])
    skip_ref[0] = m.astype(jnp.float32).T


def _tc_stage(nidx, relx, rely, relz, f2, xi2, wrep, kernel_points):
    grid = (B * N // RB,)
    fspec = pl.BlockSpec((RB, NSAMPLE), lambda i: (i, 0))
    return pl.pallas_call(
        _tc_body,
        grid=grid,
        in_specs=[
            pl.BlockSpec((RB, NSAMPLE), lambda i: (i, 0)),
            fspec, fspec, fspec,
            pl.BlockSpec((RB, SC), lambda i: (i, 0)),
            pl.BlockSpec((RB, SC), lambda i: (i, 0)),
            pl.BlockSpec((K, SC, C_OUT), lambda i: (0, 0, 0)),
            pl.BlockSpec((K, 3), lambda i: (0, 0),
                         memory_space=pltpu.SMEM),
        ],
        out_specs=[
            pl.BlockSpec((1, C_OUT, RB), lambda i: (i // NBN, 0, i % NBN)),
            pl.BlockSpec((1, C_OUT, RB), lambda i: (i // NBN, 0, i % NBN)),
        ],
        out_shape=[
            jax.ShapeDtypeStruct((B, C_OUT, N), jnp.float32),
            jax.ShapeDtypeStruct((B, C_OUT, N), jnp.float32),
        ],
    )(nidx, relx, rely, relz, f2, xi2, wrep, kernel_points)


def kernel(p, x, x_in, weights, kernel_points):
    px = p[:, :, 0].reshape(B * N)
    py = p[:, :, 1].reshape(B * N)
    pz = p[:, :, 2].reshape(B * N)
    xt = jnp.transpose(x, (0, 2, 1)).reshape(B * N, C_IN).astype(jnp.bfloat16)
    xit = jnp.transpose(x_in, (0, 2, 1)).reshape(B * N, C_IN).astype(jnp.bfloat16)
    nidx_f, relx, rely, relz, feat, xing = _sc_stage(px, py, pz, xt, xit)
    f2 = feat.reshape(B * N, SC)
    xi2 = xing.reshape(B * N, SC)
    wrep = jnp.tile(weights, (1, NSAMPLE, 1)).astype(jnp.bfloat16)
    out, skip = _tc_stage(nidx_f, relx, rely, relz, f2, xi2,
                          wrep, kernel_points)
    return out, p, skip, nidx_f.reshape(B, N, NSAMPLE)
